# baseline (speedup 1.0000x reference)
import functools

import jax
import jax.numpy as jnp
from jax import lax
from jax.experimental import pallas as pl
from jax.experimental.pallas import tpu as pltpu
from jax.experimental.pallas import tpu_sc as plsc

_NC = 2
_NS = 16
_NW = _NC * _NS

_NBUF = 4
_LAG = 2


@functools.partial(jax.jit, static_argnames=("rows", "seq", "d"))
def _emb_lookup(x, weight, *, rows, seq, d):
  rows_per_w = rows // _NW

  @functools.partial(
      pl.kernel,
      out_type=jax.ShapeDtypeStruct((rows, seq, d), jnp.float32),
      mesh=plsc.VectorSubcoreMesh(core_axis_name="c", subcore_axis_name="s"),
      scratch_types=[
          pltpu.VMEM((rows_per_w, seq), jnp.int32),
          pltpu.VMEM((_NBUF, seq, d), jnp.float32),
          [pltpu.SemaphoreType.DMA] * _NBUF,
          [pltpu.SemaphoreType.DMA] * _NBUF,
      ],
      compiler_params=pltpu.CompilerParams(use_tc_tiling_on_sc=False),
  )
  def k(table_hbm, idx_hbm, out_hbm, idx_v, rows_v, gsems, ssems):
    w = lax.axis_index("s") * _NC + lax.axis_index("c")
    base = w * rows_per_w
    pltpu.sync_copy(idx_hbm.at[pl.ds(base, rows_per_w)], idx_v)

    def wait_gather(b):
      pltpu.make_async_copy(out_hbm.at[0], rows_v.at[b], gsems[b]).wait()

    def wait_store(b):
      pltpu.make_async_copy(rows_v.at[b], out_hbm.at[0], ssems[b]).wait()

    @pl.loop(0, rows_per_w, step=_NBUF)
    def _(g):
      for b in range(_NBUF):
        c = g + b

        @pl.when(c >= _NBUF)
        def _():
          wait_store(b)

        pltpu.async_copy(table_hbm.at[idx_v.at[c]], rows_v.at[b], gsems[b])
        b2 = (b - _LAG) % _NBUF

        @pl.when(c >= _LAG)
        def _():
          wait_gather(b2)
          pltpu.async_copy(rows_v.at[b2], out_hbm.at[base + c - _LAG],
                           ssems[b2])

    for i in range(_LAG):
      c = rows_per_w - _LAG + i
      b = c % _NBUF
      wait_gather(b)
      pltpu.async_copy(rows_v.at[b], out_hbm.at[base + c], ssems[b])
    for b in range(_NBUF):
      wait_store(b)

  return k(weight, x)


def kernel(x, weight):
  rows, seq = x.shape
  d = weight.shape[-1]
  return _emb_lookup(x.astype(jnp.int32), weight, rows=rows, seq=seq, d=d)
